# baseline (speedup 1.0000x reference)
import functools

import jax
import jax.numpy as jnp
from jax import lax
from jax.experimental import pallas as pl
from jax.experimental.pallas import tpu as pltpu
from jax.experimental.pallas import tpu_sc as plsc

EMB_DIM = 64
NUM_CORES = 2
NUM_SUBCORES = 16
NUM_WORKERS = NUM_CORES * NUM_SUBCORES
HC = 5


def _make_lookup(batch: int, hist: int):
    per_worker = batch // NUM_WORKERS
    chunks = hist // HC
    mesh = plsc.VectorSubcoreMesh(core_axis_name="c", subcore_axis_name="s")

    @functools.partial(
        pl.kernel,
        mesh=mesh,
        out_type=jax.ShapeDtypeStruct((hist, EMB_DIM, batch), jnp.float32),
        scratch_types=[
            pltpu.VMEM((hist, per_worker), jnp.int32),
            pltpu.VMEM((HC * per_worker, EMB_DIM), jnp.float32),
            pltpu.VMEM((HC, EMB_DIM, per_worker), jnp.float32),
            pltpu.SemaphoreType.DMA,
        ],
        compiler_params=pltpu.CompilerParams(
            use_tc_tiling_on_sc=False, needs_layout_passes=False
        ),
    )
    def lookup(idx_hbm, table_hbm, out_hbm, idx_v, buf_g, buf_t, sem):
        wid = lax.axis_index("s") * NUM_CORES + lax.axis_index("c")
        pltpu.sync_copy(idx_hbm.at[wid], idx_v)
        lanes = lax.iota(jnp.int32, 16)

        def chunk_step(c, carry):
            gathers = [
                pltpu.async_copy(
                    table_hbm.at[idx_v.at[c * HC + hh]],
                    buf_g.at[pl.ds(hh * per_worker, per_worker)],
                    sem,
                )
                for hh in range(HC)
            ]
            for d in gathers:
                d.wait()

            def tr_step(it, carry2):
                hh = it // EMB_DIM
                e = it % EMB_DIM
                row0 = hh * per_worker
                e_vec = jnp.full((16,), e, dtype=jnp.int32)
                for g8 in range(per_worker // 16):
                    rows = jnp.full((16,), row0 + g8 * 16, dtype=jnp.int32) + lanes
                    vals = plsc.load_gather(buf_g, [rows, e_vec])
                    buf_t[hh, e, pl.ds(g8 * 16, 16)] = vals
                return carry2

            lax.fori_loop(0, HC * EMB_DIM, tr_step, 0)
            pltpu.sync_copy(
                buf_t,
                out_hbm.at[pl.ds(c * HC, HC), :, pl.ds(wid * per_worker, per_worker)],
            )
            return carry

        lax.fori_loop(0, chunks, chunk_step, 0)

    return lookup


def kernel(x, table):
    batch, hist = x.shape
    idx3d = x.reshape(NUM_WORKERS, batch // NUM_WORKERS, hist).transpose(0, 2, 1)
    out = _make_lookup(batch, hist)(idx3d, table)
    return out.transpose(2, 0, 1)


# speedup vs baseline: 1.0002x; 1.0002x over previous
import functools

import jax
import jax.numpy as jnp
from jax import lax
from jax.experimental import pallas as pl
from jax.experimental.pallas import tpu as pltpu
from jax.experimental.pallas import tpu_sc as plsc

EMB_DIM = 64
NUM_CORES = 2
NUM_SUBCORES = 16
NUM_WORKERS = NUM_CORES * NUM_SUBCORES
HC = 5


def _make_lookup(batch: int, hist: int):
    per_worker = batch // NUM_WORKERS
    chunks = hist // HC
    mesh = plsc.VectorSubcoreMesh(core_axis_name="c", subcore_axis_name="s")

    @functools.partial(
        pl.kernel,
        mesh=mesh,
        out_type=jax.ShapeDtypeStruct((hist, EMB_DIM, batch), jnp.float32),
        scratch_types=[
            pltpu.VMEM((hist, per_worker), jnp.int32),
            pltpu.VMEM((HC * per_worker, EMB_DIM), jnp.float32),
            pltpu.VMEM((HC, EMB_DIM, per_worker), jnp.float32),
            pltpu.SemaphoreType.DMA,
        ],
        compiler_params=pltpu.CompilerParams(
            use_tc_tiling_on_sc=False, needs_layout_passes=False
        ),
    )
    def lookup(idx_hbm, table_hbm, out_hbm, idx_v, buf_g, buf_t, sem):
        wid = lax.axis_index("s") * NUM_CORES + lax.axis_index("c")
        pltpu.sync_copy(idx_hbm.at[wid], idx_v)
        lanes = lax.iota(jnp.int32, 16)

        def chunk_step(c, carry):
            gathers = [
                pltpu.async_copy(
                    table_hbm.at[idx_v.at[c * HC + hh]],
                    buf_g.at[pl.ds(hh * per_worker, per_worker)],
                    sem,
                )
                for hh in range(HC)
            ]
            for d in gathers:
                d.wait()

            for hh in range(HC):
                rows_vecs = [
                    lanes + (hh * per_worker + g8 * 16)
                    for g8 in range(per_worker // 16)
                ]

                def tr_step(e, carry2, hh=hh, rows_vecs=rows_vecs):
                    e_vec = jnp.full((16,), e, dtype=jnp.int32)
                    for g8 in range(per_worker // 16):
                        vals = plsc.load_gather(buf_g, [rows_vecs[g8], e_vec])
                        buf_t[hh, e, pl.ds(g8 * 16, 16)] = vals
                    return carry2

                lax.fori_loop(0, EMB_DIM, tr_step, 0)
            pltpu.sync_copy(
                buf_t,
                out_hbm.at[pl.ds(c * HC, HC), :, pl.ds(wid * per_worker, per_worker)],
            )
            return carry

        lax.fori_loop(0, chunks, chunk_step, 0)

    return lookup


def kernel(x, table):
    batch, hist = x.shape
    idx3d = x.reshape(NUM_WORKERS, batch // NUM_WORKERS, hist).transpose(0, 2, 1)
    out = _make_lookup(batch, hist)(idx3d, table)
    return out.transpose(2, 0, 1)


# speedup vs baseline: 1.2166x; 1.2164x over previous
import functools

import jax
import jax.numpy as jnp
from jax import lax
from jax.experimental import pallas as pl
from jax.experimental.pallas import tpu as pltpu
from jax.experimental.pallas import tpu_sc as plsc

EMB_DIM = 64
NUM_CORES = 2
NUM_SUBCORES = 16
NUM_WORKERS = NUM_CORES * NUM_SUBCORES
HC = 5


def _make_lookup(batch: int, hist: int):
    per_worker = batch // NUM_WORKERS
    chunks = hist // HC
    mesh = plsc.VectorSubcoreMesh(core_axis_name="c", subcore_axis_name="s")

    @functools.partial(
        pl.kernel,
        mesh=mesh,
        out_type=jax.ShapeDtypeStruct((hist, EMB_DIM, batch), jnp.float32),
        scratch_types=[
            pltpu.VMEM((hist, per_worker), jnp.int32),
            pltpu.VMEM((HC * per_worker, EMB_DIM), jnp.float32),
            pltpu.VMEM((HC, EMB_DIM, 137), jnp.float32),
            pltpu.SemaphoreType.DMA,
        ],
        compiler_params=pltpu.CompilerParams(
            use_tc_tiling_on_sc=False, needs_layout_passes=False
        ),
    )
    def lookup(idx_hbm, table_hbm, out_hbm, idx_v, buf_g, buf_t, sem):
        wid = lax.axis_index("s") * NUM_CORES + lax.axis_index("c")
        pltpu.sync_copy(idx_hbm.at[wid], idx_v)
        lanes = lax.iota(jnp.int32, 16)

        def chunk_step(c, carry):
            gathers = [
                pltpu.async_copy(
                    table_hbm.at[idx_v.at[c * HC + hh]],
                    buf_g.at[pl.ds(hh * per_worker, per_worker)],
                    sem,
                )
                for hh in range(HC)
            ]
            for d in gathers:
                d.wait()

            e_vecs = [lanes + e0 * 16 for e0 in range(EMB_DIM // 16)]

            def tr_step(b, carry2):
                b_vec = jnp.full((16,), b, dtype=jnp.int32)
                for hh in range(HC):
                    hh_vec = jnp.full((16,), hh, dtype=jnp.int32)
                    for e0 in range(EMB_DIM // 16):
                        vals = buf_g[hh * per_worker + b, pl.ds(e0 * 16, 16)]
                        plsc.store_scatter(
                            buf_t, [hh_vec, e_vecs[e0], b_vec], vals
                        )
                return carry2

            lax.fori_loop(0, per_worker, tr_step, 0)
            pltpu.sync_copy(
                buf_t.at[:, :, pl.ds(0, per_worker)],
                out_hbm.at[pl.ds(c * HC, HC), :, pl.ds(wid * per_worker, per_worker)],
            )
            return carry

        lax.fori_loop(0, chunks, chunk_step, 0)

    return lookup


def kernel(x, table):
    batch, hist = x.shape
    idx3d = x.reshape(NUM_WORKERS, batch // NUM_WORKERS, hist).transpose(0, 2, 1)
    out = _make_lookup(batch, hist)(idx3d, table)
    return out.transpose(2, 0, 1)
